# baseline (speedup 1.0000x reference)
import jax
import jax.numpy as jnp
from jax import lax
from jax.experimental import pallas as pl
from jax.experimental.pallas import tpu as pltpu
from jax.experimental.pallas import tpu_sc as plsc

NC = 2
NS = 16
NW = NC * NS
LANES = 16
CHUNK = 128
B = 16384
B_PER_W = B // NW
NCHUNK = B_PER_W // CHUNK
TILE1D = 1024


def _bias_body(ut_hbm, mt_hbm, u_hbm, m_hbm, gb_hbm, out_hbm,
               uidx_v, midx_v, urows_v, mrows_v, gb_v, sem):
    wid = lax.axis_index("s") * NC + lax.axis_index("c")
    pltpu.sync_copy(u_hbm.at[wid], uidx_v)
    pltpu.sync_copy(m_hbm.at[wid], midx_v)
    pltpu.sync_copy(gb_hbm, gb_v)
    cu = pltpu.async_copy(ut_hbm.at[uidx_v], urows_v, sem)
    cm = pltpu.async_copy(mt_hbm.at[midx_v], mrows_v, sem)
    cu.wait()
    cm.wait()
    gbv = gb_v[...]
    for k in range(B_PER_W // LANES):
        sl = pl.ds(k * LANES, LANES)
        urows_v[sl] = urows_v[sl] + mrows_v[sl] + gbv
    pltpu.sync_copy(urows_v, out_hbm.at[wid])


@jax.jit
def _bias_sc(ut, mt, u3, m3, gb16):
    mesh = plsc.VectorSubcoreMesh(core_axis_name="c", subcore_axis_name="s")
    return pl.kernel(
        _bias_body,
        out_type=jax.ShapeDtypeStruct((NW, B_PER_W), jnp.float32),
        mesh=mesh,
        scratch_types=[
            pltpu.VMEM((B_PER_W,), jnp.int32),
            pltpu.VMEM((B_PER_W,), jnp.int32),
            pltpu.VMEM((B_PER_W,), jnp.float32),
            pltpu.VMEM((B_PER_W,), jnp.float32),
            pltpu.VMEM((LANES,), jnp.float32),
            pltpu.SemaphoreType.DMA,
        ],
    )(ut, mt, u3, m3, gb16)


def _pad_flat(table):
    v = table.shape[0]
    vp = ((v + TILE1D - 1) // TILE1D) * TILE1D
    padded = jnp.pad(table.T, ((0, 0), (0, vp - v)))
    return padded.reshape(vp)


def kernel(u, m, user_table, movie_table, global_bias):
    ut = _pad_flat(user_table)
    mt = _pad_flat(movie_table)
    u3 = u.astype(jnp.int32).reshape(NW, B_PER_W)
    m3 = m.astype(jnp.int32).reshape(NW, B_PER_W)
    gb16 = jnp.broadcast_to(global_bias.astype(jnp.float32), (LANES,))
    out = _bias_sc(ut, mt, u3, m3, gb16)
    return out.reshape(B, 1)


# speedup vs baseline: 1.1953x; 1.1953x over previous
import jax
import jax.numpy as jnp
from jax import lax
from jax.experimental import pallas as pl
from jax.experimental.pallas import tpu as pltpu
from jax.experimental.pallas import tpu_sc as plsc

NC = 2
NS = 16
NW = NC * NS
LANES = 16
CHUNK = 128
B = 16384
B_PER_W = B // NW
NCHUNK = B_PER_W // CHUNK
TILE1D = 1024


def _bias_body(ut_hbm, mt_hbm, u_hbm, m_hbm, gb_hbm, out_hbm,
               uidx_v, midx_v, urows_v, mrows_v, gb_v, sem, sem2, sem3, sem4):
    wid = lax.axis_index("s") * NC + lax.axis_index("c")
    su = pltpu.async_copy(u_hbm.at[wid], uidx_v, sem2)
    sm = pltpu.async_copy(m_hbm.at[wid], midx_v, sem3)
    sg = pltpu.async_copy(gb_hbm, gb_v, sem4)
    copies = []
    su.wait()
    for j in range(NCHUNK):
        copies.append(pltpu.async_copy(ut_hbm.at[uidx_v.at[j]], urows_v.at[j], sem))
    sm.wait()
    for j in range(NCHUNK):
        copies.append(pltpu.async_copy(mt_hbm.at[midx_v.at[j]], mrows_v.at[j], sem))
    sg.wait()
    for cp in copies:
        cp.wait()
    gbv = gb_v[...]
    for j in range(NCHUNK):
        for k in range(CHUNK // LANES):
            sl = pl.ds(k * LANES, LANES)
            urows_v[j, sl] = urows_v[j, sl] + mrows_v[j, sl] + gbv
    pltpu.sync_copy(urows_v, out_hbm.at[wid])


@jax.jit
def _bias_sc(ut, mt, u3, m3, gb16):
    mesh = plsc.VectorSubcoreMesh(core_axis_name="c", subcore_axis_name="s")
    return pl.kernel(
        _bias_body,
        out_type=jax.ShapeDtypeStruct((NW, NCHUNK, CHUNK), jnp.float32),
        mesh=mesh,
        scratch_types=[
            pltpu.VMEM((NCHUNK, CHUNK), jnp.int32),
            pltpu.VMEM((NCHUNK, CHUNK), jnp.int32),
            pltpu.VMEM((NCHUNK, CHUNK), jnp.float32),
            pltpu.VMEM((NCHUNK, CHUNK), jnp.float32),
            pltpu.VMEM((LANES,), jnp.float32),
            pltpu.SemaphoreType.DMA,
            pltpu.SemaphoreType.DMA,
            pltpu.SemaphoreType.DMA,
            pltpu.SemaphoreType.DMA,
        ],
    )(ut, mt, u3, m3, gb16)


def _pad_flat(table):
    v = table.shape[0]
    vp = ((v + TILE1D - 1) // TILE1D) * TILE1D
    padded = jnp.pad(table.T, ((0, 0), (0, vp - v)))
    return padded.reshape(vp)


def kernel(u, m, user_table, movie_table, global_bias):
    ut = _pad_flat(user_table)
    mt = _pad_flat(movie_table)
    u3 = u.astype(jnp.int32).reshape(NW, NCHUNK, CHUNK)
    m3 = m.astype(jnp.int32).reshape(NW, NCHUNK, CHUNK)
    gb16 = jnp.broadcast_to(global_bias.astype(jnp.float32), (LANES,))
    out = _bias_sc(ut, mt, u3, m3, gb16)
    return out.reshape(B, 1)
